# baseline (speedup 1.0000x reference)
import functools

import jax
import jax.numpy as jnp
from jax import lax
from jax.experimental import pallas as pl
from jax.experimental.pallas import tpu as pltpu
from jax.experimental.pallas import tpu_sc as plsc

BATCH = 16384
EMBED_DIM = 64

_NC = 2
_NS = 16
_NW = _NC * _NS
_BPW = BATCH // _NW
_G = 16
_NGRP = _BPW // _G


def _gather_kernel(idx_hbm, table_hbm, out_hbm, idx_v, buf_v, out_v,
                   sem_a, sem_b):
    wid = lax.axis_index("s") * _NC + lax.axis_index("c")
    base = wid * _BPW
    iota = lax.iota(jnp.int32, 16)

    pltpu.sync_copy(idx_hbm.at[wid], idx_v)

    def chunk_of(g):
        return idx_v[g >> 3, pl.ds((g & 7) * 16, 16)]

    def scalar_at(vec, i):
        return jnp.max(jnp.where(iota == i, vec, 0))

    def issue_group(g, p, sem):
        chunk = chunk_of(g)
        for i in range(_G):
            off = pl.multiple_of((scalar_at(chunk, i) >> 3) * 8, 8)
            pltpu.async_copy(
                table_hbm.at[pl.ds(off, 8)], buf_v.at[p, i], sem)

    def drain_group(p, sem):
        for i in range(_G):
            pltpu.make_async_copy(
                table_hbm.at[pl.ds(0, 8)], buf_v.at[p, i], sem).wait()

    def extract_group(g, p):
        chunk = chunk_of(g)
        for i in range(_G):
            r = g * _G + i
            s_vec = jnp.full((16,), scalar_at(chunk, i) & 7, jnp.int32)
            for k in range(EMBED_DIM // 16):
                val = plsc.load_gather(
                    buf_v.at[p, i], [s_vec, iota + 16 * k])
                out_v[r, pl.ds(16 * k, 16)] = val

    issue_group(0, 0, sem_a)
    issue_group(1, 1, sem_b)

    def body(gp, carry):
        g0 = gp * 2
        drain_group(0, sem_a)
        extract_group(g0, 0)

        @pl.when(g0 + 2 < _NGRP)
        def _():
            issue_group(g0 + 2, 0, sem_a)

        drain_group(1, sem_b)
        extract_group(g0 + 1, 1)

        @pl.when(g0 + 3 < _NGRP)
        def _():
            issue_group(g0 + 3, 1, sem_b)

        return carry

    lax.fori_loop(0, _NGRP // 2, body, 0)

    pltpu.sync_copy(out_v, out_hbm.at[pl.ds(base, _BPW)])


@jax.jit
def _embed_gather(idx_r, table):
    mesh = plsc.VectorSubcoreMesh(core_axis_name="c", subcore_axis_name="s")
    run = functools.partial(
        pl.kernel,
        mesh=mesh,
        out_type=jax.ShapeDtypeStruct((BATCH, EMBED_DIM), jnp.float32),
        scratch_types=[
            pltpu.VMEM((_BPW // 128, 128), jnp.int32),
            pltpu.VMEM((2, _G, 8, EMBED_DIM), jnp.float32),
            pltpu.VMEM((_BPW, EMBED_DIM), jnp.float32),
            pltpu.SemaphoreType.DMA,
            pltpu.SemaphoreType.DMA,
        ],
        compiler_params=pltpu.CompilerParams(needs_layout_passes=False),
    )(_gather_kernel)
    return run(idx_r, table)


def _transpose_body(t_ref, o_ref):
    o_ref[...] = t_ref[...].T


_VBLK = 512


@jax.jit
def _transpose_tc(table_t):
    vocab = table_t.shape[1]
    grid = (vocab + _VBLK - 1) // _VBLK
    return pl.pallas_call(
        _transpose_body,
        grid=(grid,),
        in_specs=[pl.BlockSpec((EMBED_DIM, _VBLK), lambda i: (0, i))],
        out_specs=pl.BlockSpec((_VBLK, EMBED_DIM), lambda i: (i, 0)),
        out_shape=jax.ShapeDtypeStruct((vocab, EMBED_DIM), jnp.float32),
    )(table_t)


def kernel(in_idxs, embed_in):
    idx_r = in_idxs.astype(jnp.int32).reshape(_NW, _BPW // 128, 128)
    table_rm = _transpose_tc(embed_in.T)
    return _embed_gather(idx_r, table_rm)


# speedup vs baseline: 3.2031x; 3.2031x over previous
import functools

import jax
import jax.numpy as jnp
from jax import lax
from jax.experimental import pallas as pl
from jax.experimental.pallas import tpu as pltpu
from jax.experimental.pallas import tpu_sc as plsc

BATCH = 16384
EMBED_DIM = 64

_NC = 2
_NS = 16
_NW = _NC * _NS
_BPW = BATCH // _NW
_G = 16
_NGRP = _BPW // _G


def _gather_kernel(idx_hbm, table_hbm, out_hbm, idx_v, buf_v, out_v,
                   sem_a, sem_b):
    wid = lax.axis_index("s") * _NC + lax.axis_index("c")
    base = wid * _BPW
    iota = lax.iota(jnp.int32, 16)

    pltpu.sync_copy(idx_hbm.at[wid], idx_v)

    def chunk_of(g):
        return idx_v[g >> 3, pl.ds((g & 7) * 16, 16)]

    def scalar_at(vec, i):
        return jnp.max(jnp.where(iota == i, vec, 0))

    def issue_group(g, p, sem):
        chunk = chunk_of(g)
        for i in range(_G):
            off = pl.multiple_of((scalar_at(chunk, i) >> 3) * 8, 8)
            pltpu.async_copy(
                table_hbm.at[pl.ds(off, 8)], buf_v.at[p, i], sem)

    def drain_group(p, sem):
        for i in range(_G):
            pltpu.make_async_copy(
                table_hbm.at[pl.ds(0, 8)], buf_v.at[p, i], sem).wait()

    def extract_group(g, p):
        j_vec = jnp.full((16,), g >> 3, jnp.int32)
        for i in range(_G):
            r = g * _G + i
            col = jnp.full((16,), (g & 7) * 16 + i, jnp.int32)
            s_vec = plsc.load_gather(idx_v, [j_vec, col]) & 7
            for k in range(EMBED_DIM // 16):
                val = plsc.load_gather(
                    buf_v.at[p, i], [s_vec, iota + 16 * k])
                out_v[r, pl.ds(16 * k, 16)] = val

    issue_group(0, 0, sem_a)
    issue_group(1, 1, sem_b)

    def body(gp, carry):
        g0 = gp * 2
        drain_group(0, sem_a)
        extract_group(g0, 0)

        @pl.when(g0 + 2 < _NGRP)
        def _():
            issue_group(g0 + 2, 0, sem_a)

        drain_group(1, sem_b)
        extract_group(g0 + 1, 1)

        @pl.when(g0 + 3 < _NGRP)
        def _():
            issue_group(g0 + 3, 1, sem_b)

        return carry

    lax.fori_loop(0, _NGRP // 2, body, 0)

    pltpu.sync_copy(out_v, out_hbm.at[pl.ds(base, _BPW)])


@jax.jit
def _embed_gather(idx_r, table):
    mesh = plsc.VectorSubcoreMesh(core_axis_name="c", subcore_axis_name="s")
    run = functools.partial(
        pl.kernel,
        mesh=mesh,
        out_type=jax.ShapeDtypeStruct((BATCH, EMBED_DIM), jnp.float32),
        scratch_types=[
            pltpu.VMEM((_BPW // 128, 128), jnp.int32),
            pltpu.VMEM((2, _G, 8, EMBED_DIM), jnp.float32),
            pltpu.VMEM((_BPW, EMBED_DIM), jnp.float32),
            pltpu.SemaphoreType.DMA,
            pltpu.SemaphoreType.DMA,
        ],
        compiler_params=pltpu.CompilerParams(needs_layout_passes=False),
    )(_gather_kernel)
    return run(idx_r, table)


def kernel(in_idxs, embed_in):
    idx_r = in_idxs.astype(jnp.int32).reshape(_NW, _BPW // 128, 128)
    return _embed_gather(idx_r, embed_in)
